# baseline (speedup 1.0000x reference)
import functools

import jax
import jax.numpy as jnp
from jax import lax
from jax.experimental import pallas as pl
from jax.experimental.pallas import tpu as pltpu
from jax.experimental.pallas import tpu_sc as plsc

_EMBED_SCALE = 11.313708498984761

_NC = 2
_NS = 16
_NW = _NC * _NS
_G = 128
_NBUF = 5
_PREF = 4


def _make_sc_gather(n_rows, vocab, dim):
    per_w = n_rows // _NW
    ng = per_w // _G

    mesh = plsc.VectorSubcoreMesh(core_axis_name="c", subcore_axis_name="s")

    @functools.partial(
        pl.kernel,
        mesh=mesh,
        out_type=jax.ShapeDtypeStruct((n_rows, dim), jnp.float32),
        scratch_types=[
            pltpu.VMEM((ng, _G), jnp.int32),
            [pltpu.VMEM((_G, dim), jnp.float32) for _ in range(_NBUF)],
            [pltpu.SemaphoreType.DMA for _ in range(_NBUF)],
            [pltpu.SemaphoreType.DMA for _ in range(_NBUF)],
        ],
    )
    def sc_kernel(ids_hbm, table_hbm, out_hbm, idx_v, bufs, sgs, sss):
        wid = lax.axis_index("s") * _NC + lax.axis_index("c")
        base = wid * per_w

        def gather(g, b):
            return pltpu.make_async_copy(
                table_hbm.at[idx_v.at[g]], bufs[b], sgs[b])

        def scatter(g, b):
            return pltpu.make_async_copy(
                bufs[b], out_hbm.at[pl.ds(base + g * _G, _G)], sss[b])

        pltpu.sync_copy(ids_hbm.at[wid, pl.ds(0, 8)], idx_v.at[pl.ds(0, 8)])
        for b in range(_PREF):
            gather(b, b).start()
        pltpu.sync_copy(ids_hbm.at[wid, pl.ds(8, ng - 8)],
                        idx_v.at[pl.ds(8, ng - 8)])

        def outer(i, carry):
            for b in range(_NBUF):
                g = i * _NBUF + b
                gather(g, b).wait()

                def scale_row(r, c2):
                    for c in range(dim // 16):
                        sl = pl.ds(c * 16, 16)
                        bufs[b][r, sl] = bufs[b][r, sl] * _EMBED_SCALE
                    return c2
                lax.fori_loop(0, _G, scale_row, 0, unroll=4)

                scatter(g, b).start()

                bp = (b + _PREF) % _NBUF

                @pl.when(g + _PREF - _NBUF >= 0)
                def _():
                    scatter(g + _PREF - _NBUF, bp).wait()

                @pl.when(g + _PREF < ng)
                def _():
                    gather(g + _PREF, bp).start()
            return carry

        lax.fori_loop(0, ng // _NBUF, outer, 0)

        for g in range(ng - (_NBUF - _PREF), ng):
            scatter(g, g % _NBUF).wait()

    return sc_kernel


def _tc_scale_body(x_ref, o_ref):
    o_ref[...] = x_ref[...] * _EMBED_SCALE


def _tc_scale(flat, blk=8192):
    n, dim = flat.shape
    return pl.pallas_call(
        _tc_scale_body,
        grid=(n // blk,),
        in_specs=[pl.BlockSpec((blk, dim), lambda i: (i, 0))],
        out_specs=pl.BlockSpec((blk, dim), lambda i: (i, 0)),
        out_shape=jax.ShapeDtypeStruct((n, dim), jnp.float32),
    )(flat)


def kernel(input_ids, weight):
    batch, seq = input_ids.shape
    vocab, dim = weight.shape
    n_rows = batch * seq
    assert n_rows % (_NW * _G) == 0 and dim % 16 == 0

    ids_t = jnp.transpose(input_ids).reshape(_NW, n_rows // (_NW * _G), _G)
    flat = _make_sc_gather(n_rows, vocab, dim)(ids_t, weight)
    return jnp.transpose(flat.reshape(seq, batch, dim), (1, 0, 2))


# speedup vs baseline: 1.0030x; 1.0030x over previous
import functools

import jax
import jax.numpy as jnp
from jax import lax
from jax.experimental import pallas as pl
from jax.experimental.pallas import tpu as pltpu
from jax.experimental.pallas import tpu_sc as plsc

_EMBED_SCALE = 11.313708498984761

_NC = 2
_NS = 16
_NW = _NC * _NS
_G = 128
_NBUF = 5
_PREF = 4


def _make_sc_gather(n_rows, dim):
    per_w = n_rows // _NW
    ng = per_w // _G

    mesh = plsc.VectorSubcoreMesh(core_axis_name="c", subcore_axis_name="s")

    @functools.partial(
        pl.kernel,
        mesh=mesh,
        out_type=jax.ShapeDtypeStruct((n_rows, dim), jnp.float32),
        scratch_types=[
            pltpu.VMEM((ng, _G), jnp.int32),
            [pltpu.VMEM((_G, dim), jnp.float32) for _ in range(_NBUF)],
            [pltpu.SemaphoreType.DMA for _ in range(_NBUF)],
            [pltpu.SemaphoreType.DMA for _ in range(_NBUF)],
        ],
    )
    def sc_kernel(ids_hbm, table_hbm, out_hbm, idx_v, bufs, sgs, sss):
        wid = lax.axis_index("s") * _NC + lax.axis_index("c")
        base = wid * per_w

        def gather(g, b):
            return pltpu.make_async_copy(
                table_hbm.at[idx_v.at[g]], bufs[b], sgs[b])

        def scatter(g, b):
            return pltpu.make_async_copy(
                bufs[b], out_hbm.at[pl.ds(base + g * _G, _G)], sss[b])

        pltpu.sync_copy(ids_hbm.at[wid, pl.ds(0, 8)], idx_v.at[pl.ds(0, 8)])
        for b in range(_PREF):
            gather(b, b).start()
        pltpu.sync_copy(ids_hbm.at[wid, pl.ds(8, ng - 8)],
                        idx_v.at[pl.ds(8, ng - 8)])

        def outer(i, carry):
            for b in range(_NBUF):
                g = i * _NBUF + b
                gather(g, b).wait()

                def scale_row(r, c2):
                    for c in range(dim // 16):
                        sl = pl.ds(c * 16, 16)
                        bufs[b][r, sl] = bufs[b][r, sl] * _EMBED_SCALE
                    return c2
                lax.fori_loop(0, _G, scale_row, 0, unroll=4)

                scatter(g, b).start()

                bp = (b + _PREF) % _NBUF

                @pl.when(g + _PREF - _NBUF >= 0)
                def _():
                    scatter(g + _PREF - _NBUF, bp).wait()

                @pl.when(g + _PREF < ng)
                def _():
                    gather(g + _PREF, bp).start()
            return carry

        lax.fori_loop(0, ng // _NBUF, outer, 0)

        for g in range(ng - (_NBUF - _PREF), ng):
            scatter(g, g % _NBUF).wait()

    return sc_kernel


def kernel(input_ids, weight):
    batch, seq = input_ids.shape
    _, dim = weight.shape
    n_rows = batch * seq
    assert n_rows % (_NW * _G) == 0 and dim % 16 == 0

    ids_t = jnp.transpose(input_ids).reshape(_NW, n_rows // (_NW * _G), _G)
    flat = _make_sc_gather(n_rows, dim)(ids_t, weight)
    return jnp.transpose(flat.reshape(seq, batch, dim), (1, 0, 2))
